# baseline (device time: 212682 ns/iter reference)
import jax
import jax.numpy as jnp
from jax import lax
from jax.experimental import pallas as pl
from jax.experimental.pallas import tpu as pltpu


def kernel(x, pi):
    x = x.astype(jnp.bfloat16)

    def body(pi_ref, x_ref, out_ref, send_sem, recv_sem):
        my_x = lax.axis_index("x")
        my_y = lax.axis_index("y")
        dst_x = pi_ref[my_x]
        src_x = jnp.where(pi_ref[0] == my_x, 0, 1)

        barrier_sem = pltpu.get_barrier_semaphore()
        pl.semaphore_signal(
            barrier_sem,
            inc=1,
            device_id=(src_x, my_y),
            device_id_type=pl.DeviceIdType.MESH,
        )
        pl.semaphore_wait(barrier_sem, 1)

        rdma = pltpu.make_async_remote_copy(
            src_ref=x_ref,
            dst_ref=out_ref,
            send_sem=send_sem,
            recv_sem=recv_sem,
            device_id=(dst_x, my_y),
            device_id_type=pl.DeviceIdType.MESH,
        )
        rdma.start()
        rdma.wait()

    return pl.pallas_call(
        body,
        out_shape=jax.ShapeDtypeStruct(x.shape, x.dtype),
        in_specs=[
            pl.BlockSpec(memory_space=pltpu.SMEM),
            pl.BlockSpec(memory_space=pltpu.VMEM),
        ],
        out_specs=pl.BlockSpec(memory_space=pltpu.VMEM),
        scratch_shapes=[
            pltpu.SemaphoreType.DMA,
            pltpu.SemaphoreType.DMA,
        ],
        compiler_params=pltpu.CompilerParams(collective_id=0),
    )(pi, x)


# device time: 193106 ns/iter; 1.1014x vs baseline; 1.1014x over previous
import jax
import jax.numpy as jnp
from jax import lax
from jax.experimental import pallas as pl
from jax.experimental.pallas import tpu as pltpu

N_CHUNKS = 8


def kernel(x, pi):
    _, m, n = x.shape
    rows = m // N_CHUNKS

    def body(pi_ref, x_ref, out_ref, load_buf, send_buf,
             load_sems, send_sems, recv_sems):
        my_x = lax.axis_index("x")
        my_y = lax.axis_index("y")
        dst_x = pi_ref[my_x]
        src_x = jnp.where(pi_ref[0] == my_x, 0, 1)

        barrier_sem = pltpu.get_barrier_semaphore()
        pl.semaphore_signal(
            barrier_sem,
            inc=1,
            device_id=(src_x, my_y),
            device_id_type=pl.DeviceIdType.MESH,
        )
        pl.semaphore_wait(barrier_sem, 1)

        def load(i, slot):
            cp = pltpu.make_async_copy(
                x_ref.at[0, pl.ds(i * rows, rows), :],
                load_buf.at[slot],
                load_sems.at[slot],
            )
            cp.start()
            return cp

        def chunk_rdma(i):
            return pltpu.make_async_remote_copy(
                src_ref=send_buf.at[pl.ds(i * rows, rows), :],
                dst_ref=out_ref.at[0, pl.ds(i * rows, rows), :],
                send_sem=send_sems.at[i],
                recv_sem=recv_sems.at[i],
                device_id=(dst_x, my_y),
                device_id_type=pl.DeviceIdType.MESH,
            )

        load(0, 0)
        rdmas = []
        for i in range(N_CHUNKS):
            slot = i % 2
            if i + 1 < N_CHUNKS:
                load(i + 1, (i + 1) % 2)
            pltpu.make_async_copy(
                x_ref.at[0, pl.ds(i * rows, rows), :],
                load_buf.at[slot],
                load_sems.at[slot],
            ).wait()
            send_buf[pl.ds(i * rows, rows), :] = load_buf[slot].astype(
                jnp.bfloat16
            )
            rdma = chunk_rdma(i)
            rdma.start()
            rdmas.append(rdma)

        for rdma in rdmas:
            rdma.wait_send()
            rdma.wait_recv()

    return pl.pallas_call(
        body,
        out_shape=jax.ShapeDtypeStruct((1, m, n), jnp.bfloat16),
        in_specs=[
            pl.BlockSpec(memory_space=pltpu.SMEM),
            pl.BlockSpec(memory_space=pl.ANY),
        ],
        out_specs=pl.BlockSpec(memory_space=pl.ANY),
        scratch_shapes=[
            pltpu.VMEM((2, rows, n), jnp.float32),
            pltpu.VMEM((m, n), jnp.bfloat16),
            pltpu.SemaphoreType.DMA((2,)),
            pltpu.SemaphoreType.DMA((N_CHUNKS,)),
            pltpu.SemaphoreType.DMA((N_CHUNKS,)),
        ],
        compiler_params=pltpu.CompilerParams(collective_id=0),
    )(pi, x)


# device time: 115847 ns/iter; 1.8359x vs baseline; 1.6669x over previous
import jax
import jax.numpy as jnp
from jax import lax
from jax.experimental import pallas as pl
from jax.experimental.pallas import tpu as pltpu

N_CHUNKS = 8


def kernel(x, pi):
    _, m, n = x.shape
    half = m // 2
    rows = half // N_CHUNKS

    def body(pi_ref, x_ref, out_ref, load_buf, send_buf,
             load_sems, xsend_sems, xrecv_sems, fsend_sems, frecv_sems):
        my_x = lax.axis_index("x")
        my_y = lax.axis_index("y")
        dst_x = pi_ref[my_x]
        src_x = jnp.where(pi_ref[0] == my_x, 0, 1)
        half_base = my_y * half

        barrier_sem = pltpu.get_barrier_semaphore()
        pl.semaphore_signal(
            barrier_sem,
            inc=1,
            device_id=(src_x, my_y),
            device_id_type=pl.DeviceIdType.MESH,
        )
        pl.semaphore_signal(
            barrier_sem,
            inc=1,
            device_id=(my_x, 1 - my_y),
            device_id_type=pl.DeviceIdType.MESH,
        )
        pl.semaphore_wait(barrier_sem, 2)

        def load(c, slot):
            return pltpu.make_async_copy(
                x_ref.at[0, pl.ds(half_base + c * rows, rows), :],
                load_buf.at[slot],
                load_sems.at[slot],
            )

        load(0, 0).start()
        x_rdmas = []
        for c in range(N_CHUNKS):
            slot = c % 2
            if c + 1 < N_CHUNKS:
                load(c + 1, (c + 1) % 2).start()
            load(c, slot).wait()
            send_buf[pl.ds(c * rows, rows), :] = load_buf[slot].astype(
                jnp.bfloat16
            )
            rdma = pltpu.make_async_remote_copy(
                src_ref=send_buf.at[pl.ds(c * rows, rows), :],
                dst_ref=out_ref.at[0, pl.ds(half_base + c * rows, rows), :],
                send_sem=xsend_sems.at[c],
                recv_sem=xrecv_sems.at[c],
                device_id=(dst_x, my_y),
                device_id_type=pl.DeviceIdType.MESH,
            )
            rdma.start()
            x_rdmas.append(rdma)

        fwds = []
        for c in range(N_CHUNKS):
            x_rdmas[c].wait_recv()
            fwd = pltpu.make_async_remote_copy(
                src_ref=out_ref.at[0, pl.ds(half_base + c * rows, rows), :],
                dst_ref=out_ref.at[0, pl.ds(half_base + c * rows, rows), :],
                send_sem=fsend_sems.at[c],
                recv_sem=frecv_sems.at[c],
                device_id=(my_x, 1 - my_y),
                device_id_type=pl.DeviceIdType.MESH,
            )
            fwd.start()
            fwds.append(fwd)

        for c in range(N_CHUNKS):
            x_rdmas[c].wait_send()
            fwds[c].wait_send()
            fwds[c].wait_recv()

    return pl.pallas_call(
        body,
        out_shape=jax.ShapeDtypeStruct((1, m, n), jnp.bfloat16),
        in_specs=[
            pl.BlockSpec(memory_space=pltpu.SMEM),
            pl.BlockSpec(memory_space=pl.ANY),
        ],
        out_specs=pl.BlockSpec(memory_space=pl.ANY),
        scratch_shapes=[
            pltpu.VMEM((2, rows, n), jnp.float32),
            pltpu.VMEM((half, n), jnp.bfloat16),
            pltpu.SemaphoreType.DMA((2,)),
            pltpu.SemaphoreType.DMA((N_CHUNKS,)),
            pltpu.SemaphoreType.DMA((N_CHUNKS,)),
            pltpu.SemaphoreType.DMA((N_CHUNKS,)),
            pltpu.SemaphoreType.DMA((N_CHUNKS,)),
        ],
        compiler_params=pltpu.CompilerParams(collective_id=0),
    )(pi, x)


# device time: 113208 ns/iter; 1.8787x vs baseline; 1.0233x over previous
import jax
import jax.numpy as jnp
from jax import lax
from jax.experimental import pallas as pl
from jax.experimental.pallas import tpu as pltpu

N_CHUNKS = 16


def kernel(x, pi):
    _, m, n = x.shape
    half = m // 2
    rows = half // N_CHUNKS

    def body(pi_ref, x_ref, out_ref, load_buf, send_buf,
             load_sems, xsend_sems, xrecv_sems, fsend_sems, frecv_sems):
        my_x = lax.axis_index("x")
        my_y = lax.axis_index("y")
        dst_x = pi_ref[my_x]
        src_x = jnp.where(pi_ref[0] == my_x, 0, 1)
        half_base = my_y * half

        barrier_sem = pltpu.get_barrier_semaphore()
        pl.semaphore_signal(
            barrier_sem,
            inc=1,
            device_id=(src_x, my_y),
            device_id_type=pl.DeviceIdType.MESH,
        )
        pl.semaphore_signal(
            barrier_sem,
            inc=1,
            device_id=(my_x, 1 - my_y),
            device_id_type=pl.DeviceIdType.MESH,
        )
        pl.semaphore_wait(barrier_sem, 2)

        def load(c, slot):
            return pltpu.make_async_copy(
                x_ref.at[0, pl.ds(half_base + c * rows, rows), :],
                load_buf.at[slot],
                load_sems.at[slot],
            )

        load(0, 0).start()
        x_rdmas = []
        for c in range(N_CHUNKS):
            slot = c % 2
            if c + 1 < N_CHUNKS:
                load(c + 1, (c + 1) % 2).start()
            load(c, slot).wait()
            send_buf[pl.ds(c * rows, rows), :] = load_buf[slot].astype(
                jnp.bfloat16
            )
            rdma = pltpu.make_async_remote_copy(
                src_ref=send_buf.at[pl.ds(c * rows, rows), :],
                dst_ref=out_ref.at[0, pl.ds(half_base + c * rows, rows), :],
                send_sem=xsend_sems.at[c],
                recv_sem=xrecv_sems.at[c],
                device_id=(dst_x, my_y),
                device_id_type=pl.DeviceIdType.MESH,
            )
            rdma.start()
            x_rdmas.append(rdma)

        fwds = []
        for c in range(N_CHUNKS):
            x_rdmas[c].wait_recv()
            fwd = pltpu.make_async_remote_copy(
                src_ref=out_ref.at[0, pl.ds(half_base + c * rows, rows), :],
                dst_ref=out_ref.at[0, pl.ds(half_base + c * rows, rows), :],
                send_sem=fsend_sems.at[c],
                recv_sem=frecv_sems.at[c],
                device_id=(my_x, 1 - my_y),
                device_id_type=pl.DeviceIdType.MESH,
            )
            fwd.start()
            fwds.append(fwd)

        for c in range(N_CHUNKS):
            x_rdmas[c].wait_send()
            fwds[c].wait_send()
            fwds[c].wait_recv()

    return pl.pallas_call(
        body,
        out_shape=jax.ShapeDtypeStruct((1, m, n), jnp.bfloat16),
        in_specs=[
            pl.BlockSpec(memory_space=pltpu.SMEM),
            pl.BlockSpec(memory_space=pl.ANY),
        ],
        out_specs=pl.BlockSpec(memory_space=pl.ANY),
        scratch_shapes=[
            pltpu.VMEM((2, rows, n), jnp.float32),
            pltpu.VMEM((half, n), jnp.bfloat16),
            pltpu.SemaphoreType.DMA((2,)),
            pltpu.SemaphoreType.DMA((N_CHUNKS,)),
            pltpu.SemaphoreType.DMA((N_CHUNKS,)),
            pltpu.SemaphoreType.DMA((N_CHUNKS,)),
            pltpu.SemaphoreType.DMA((N_CHUNKS,)),
        ],
        compiler_params=pltpu.CompilerParams(collective_id=0),
    )(pi, x)
